# Initial kernel scaffold; baseline (speedup 1.0000x reference)
import jax
import jax.numpy as jnp
from jax.experimental import pallas as pl


def kernel(inputs, edge_index, W1_1, b1_1, W2_1, b2_1, W1_2, b1_2, W2_2, b2_2):
    raise NotImplementedError("write your pallas kernel here")


# speedup vs baseline: 16.1814x; 16.1814x over previous
import functools

import jax
import jax.numpy as jnp
from jax import lax
from jax.experimental import pallas as pl
from jax.experimental.pallas import tpu as pltpu
from jax.experimental.pallas import tpu_sc as plsc

N_NODES = 10000
NP = 10240
E = 320000
EP = 327680
NW = 32
EW = EP // NW
CHUNK = 128
NCH = EW // CHUNK
F = 128
FW = 144
RB = 256

_MESH = dict(core_axis_name="c", subcore_axis_name="s")


def _sc_degrees(src3, dst3, zrow):
    @functools.partial(
        pl.kernel,
        out_type=jax.ShapeDtypeStruct((2, 2, NP), jnp.float32),
        mesh=plsc.VectorSubcoreMesh(**_MESH),
        scratch_types=[
            pltpu.VMEM((NCH, CHUNK), jnp.int32),
            pltpu.VMEM((NCH, CHUNK), jnp.int32),
            pltpu.VMEM((CHUNK,), jnp.float32),
            pltpu.VMEM_SHARED((NP,), jnp.float32),
            pltpu.VMEM_SHARED((NP,), jnp.float32),
        ],
    )
    def deg_kernel(src_hbm, dst_hbm, zrow_hbm, out_hbm, src_v, dst_v, ones_v,
                   acc_o, acc_i):
        cid = lax.axis_index("c")
        sid = lax.axis_index("s")
        wid = sid * 2 + cid
        seg = sid * (NP // 16)
        pltpu.sync_copy(zrow_hbm, acc_o.at[pl.ds(seg, NP // 16)])
        pltpu.sync_copy(zrow_hbm, acc_i.at[pl.ds(seg, NP // 16)])
        for k in range(CHUNK // 16):
            ones_v[pl.ds(k * 16, 16)] = jnp.ones((16,), jnp.float32)
        pltpu.sync_copy(src_hbm.at[wid], src_v)
        pltpu.sync_copy(dst_hbm.at[wid], dst_v)
        plsc.subcore_barrier()

        def body(c, carry):
            pltpu.sync_copy(ones_v, acc_o.at[src_v.at[c]], add=True)
            pltpu.sync_copy(ones_v, acc_i.at[dst_v.at[c]], add=True)
            return carry

        lax.fori_loop(0, NCH, body, 0)
        plsc.subcore_barrier()
        pltpu.sync_copy(acc_o.at[pl.ds(seg, NP // 16)],
                        out_hbm.at[cid, 0, pl.ds(seg, NP // 16)])
        pltpu.sync_copy(acc_i.at[pl.ds(seg, NP // 16)],
                        out_hbm.at[cid, 1, pl.ds(seg, NP // 16)])

    return deg_kernel(src3, dst3, zrow)


def _sc_gather_scatter(table, src3, dst3, zblk):
    @functools.partial(
        pl.kernel,
        out_type=jax.ShapeDtypeStruct((2, NP, FW), jnp.float32),
        mesh=plsc.VectorSubcoreMesh(**_MESH),
        compiler_params=pltpu.CompilerParams(use_tc_tiling_on_sc=False),
        scratch_types=[
            pltpu.VMEM((NCH, CHUNK), jnp.int32),
            pltpu.VMEM((NCH, CHUNK), jnp.int32),
            pltpu.VMEM((CHUNK, FW), jnp.float32),
            pltpu.VMEM_SHARED((NP, FW), jnp.float32),
            pltpu.SemaphoreType.DMA,
        ],
    )
    def gs_kernel(table_hbm, src_hbm, dst_hbm, zblk_hbm, out_hbm, src_v, dst_v,
                  rows_v, acc, sem):
        cid = lax.axis_index("c")
        sid = lax.axis_index("s")
        wid = sid * 2 + cid
        rows_per_tile = NP // 16
        seg = sid * rows_per_tile
        pltpu.sync_copy(zblk_hbm, acc.at[pl.ds(seg, rows_per_tile)])
        pltpu.sync_copy(src_hbm.at[wid], src_v)
        pltpu.sync_copy(dst_hbm.at[wid], dst_v)
        plsc.subcore_barrier()

        def body(c, carry):
            pltpu.async_copy(table_hbm.at[src_v.at[c]], rows_v, sem).wait()
            pltpu.sync_copy(rows_v, acc.at[dst_v.at[c]], add=True)
            return carry

        lax.fori_loop(0, NCH, body, 0)
        plsc.subcore_barrier()
        pltpu.sync_copy(acc.at[pl.ds(seg, rows_per_tile)],
                        out_hbm.at[cid, pl.ds(seg, rows_per_tile)])

    return gs_kernel(table, src3, dst3, zblk)


def _tc_prep(x_pad, deg_o2):
    def prep_body(x_ref, d_ref, t_ref):
        u = lax.rsqrt(jnp.maximum(d_ref[0, :] + d_ref[1, :], 1.0))
        ux = x_ref[...] * u[:, None]
        t_ref[...] = jnp.concatenate(
            [ux, u[:, None], jnp.zeros((RB, FW - F - 1), jnp.float32)], axis=1)

    return pl.pallas_call(
        prep_body,
        grid=(NP // RB,),
        in_specs=[
            pl.BlockSpec((RB, F), lambda i: (i, 0)),
            pl.BlockSpec((2, RB), lambda i: (0, i)),
        ],
        out_specs=pl.BlockSpec((RB, FW), lambda i: (i, 0)),
        out_shape=jax.ShapeDtypeStruct((NP, FW), jnp.float32),
    )(x_pad, deg_o2)


def _tc_layer(acc0, acc1, x_pad, deg_o2, deg_i2, W1, b1, W2, b2, first):
    def layer_body(a0_ref, a1_ref, x_ref, do_ref, di_ref, w1_ref, b1_ref,
                   w2_ref, b2_ref, *outs):
        v = lax.rsqrt(jnp.maximum(di_ref[0, :] + di_ref[1, :], 1.0))
        asum = a0_ref[:, :F] + a1_ref[:, :F]
        ssum = a0_ref[:, F] + a1_ref[:, F]
        a = asum * v[:, None]
        s = ssum * v
        x = x_ref[...]
        z = (jnp.dot(a + x, w1_ref[...], preferred_element_type=jnp.float32)
             + jnp.dot(x * a, w2_ref[...], preferred_element_type=jnp.float32)
             + b1_ref[...] * (1.0 + s)[:, None]
             + b2_ref[...] * s[:, None])
        if first:
            h = jnp.maximum(z, 0.0)
            outs[0][...] = h
            u = lax.rsqrt(jnp.maximum(do_ref[0, :] + do_ref[1, :], 1.0))
            outs[1][...] = jnp.concatenate(
                [h * u[:, None], u[:, None],
                 jnp.zeros((RB, FW - F - 1), jnp.float32)], axis=1)
        else:
            outs[0][...] = jnp.maximum(z, 0.0) + 0.2 * jnp.minimum(z, 0.0)

    out_shape = [jax.ShapeDtypeStruct((NP, F), jnp.float32)]
    out_specs = [pl.BlockSpec((RB, F), lambda i: (i, 0))]
    if first:
        out_shape.append(jax.ShapeDtypeStruct((NP, FW), jnp.float32))
        out_specs.append(pl.BlockSpec((RB, FW), lambda i: (i, 0)))

    return pl.pallas_call(
        layer_body,
        grid=(NP // RB,),
        in_specs=[
            pl.BlockSpec((RB, FW), lambda i: (i, 0)),
            pl.BlockSpec((RB, FW), lambda i: (i, 0)),
            pl.BlockSpec((RB, F), lambda i: (i, 0)),
            pl.BlockSpec((2, RB), lambda i: (0, i)),
            pl.BlockSpec((2, RB), lambda i: (0, i)),
            pl.BlockSpec((F, F), lambda i: (0, 0)),
            pl.BlockSpec((1, F), lambda i: (0, 0)),
            pl.BlockSpec((F, F), lambda i: (0, 0)),
            pl.BlockSpec((1, F), lambda i: (0, 0)),
        ],
        out_specs=out_specs,
        out_shape=out_shape,
    )(acc0, acc1, x_pad, deg_o2, deg_i2, W1, b1.reshape(1, F), W2,
      b2.reshape(1, F))


def kernel(inputs, edge_index, W1_1, b1_1, W2_1, b2_1, W1_2, b1_2, W2_2, b2_2):
    x = inputs
    src = edge_index[0]
    dst = edge_index[1]
    pad_ids = (N_NODES
               + jnp.arange(EP - E, dtype=jnp.int32) % (NP - N_NODES))
    src3 = jnp.concatenate([src, pad_ids]).reshape(NW, NCH, CHUNK)
    dst3 = jnp.concatenate([dst, pad_ids]).reshape(NW, NCH, CHUNK)
    zrow = jnp.zeros((NP // 16,), jnp.float32)
    zblk = jnp.zeros((NP // 16, FW), jnp.float32)
    x_pad = jnp.pad(x, ((0, NP - N_NODES), (0, 0)))

    degs = _sc_degrees(src3, dst3, zrow)
    deg_o2 = degs[:, 0]
    deg_i2 = degs[:, 1]

    table1 = _tc_prep(x_pad, deg_o2)
    acc1 = _sc_gather_scatter(table1, src3, dst3, zblk)
    h1, table2 = _tc_layer(acc1[0], acc1[1], x_pad, deg_o2, deg_i2,
                           W1_1, b1_1, W2_1, b2_1, first=True)
    acc2 = _sc_gather_scatter(table2, src3, dst3, zblk)
    (out,) = _tc_layer(acc2[0], acc2[1], h1, deg_o2, deg_i2,
                       W1_2, b1_2, W2_2, b2_2, first=False)
    return out[:N_NODES]
